# baseline (speedup 1.0000x reference)
import jax
import jax.numpy as jnp
from jax import lax
from jax.experimental import pallas as pl
from jax.experimental.pallas import tpu as pltpu
from jax.experimental.pallas import tpu_sc as plsc

N_NODES = 10000
N_EDGES = 320000
D_IN = 128
D_OUT = 256

NC = 2
NS = 16
NW = NC * NS
E_PER_W = N_EDGES // NW
CHUNK = 128
EPT = 10240
N_CHUNKS = EPT // CHUNK
PAIRS = N_CHUNKS // 2
N_PAD = 10240
ROWS_PER_TILE = N_PAD // NS
ZROWS = 128


DEG_W = 128


def _fill_16lane(ref, nrow, ncol, value):
    nv = ncol // 16
    def row(r, _):
        def col(j, _):
            ref[r, pl.ds(j * 16, 16)] = jnp.full((16,), value, jnp.float32)
            return 0
        return lax.fori_loop(0, nv, col, 0)
    lax.fori_loop(0, nrow, row, 0)


def _sc_body(src_hbm, dst_hbm, x_hbm, agg_out,
             sidx0, sidx1, didx0, didx1, rows0, rows1, agg_sh,
             sem_g0, sem_g1, sem_s0, sem_s1):
    c = lax.axis_index("c")
    s = lax.axis_index("s")
    wid = c * NS + s
    tile_base = wid * EPT

    _fill_16lane(rows0, CHUNK, D_IN, 0.0)
    base_row = s * ROWS_PER_TILE
    for k in range(ROWS_PER_TILE // ZROWS):
        pltpu.sync_copy(rows0, agg_sh.at[pl.ds(base_row + k * ZROWS, ZROWS)])
    plsc.subcore_barrier()

    def chunk(i, _):
        base = tile_base + i * CHUNK
        pltpu.sync_copy(src_hbm.at[pl.ds(base, CHUNK)], sidx0)
        pltpu.async_copy(x_hbm.at[sidx0], rows0, sem_g0).wait()
        pltpu.sync_copy(dst_hbm.at[pl.ds(base, CHUNK)], didx0)
        pltpu.sync_copy(rows0, agg_sh.at[didx0], add=True)
        return 0

    lax.fori_loop(0, N_CHUNKS, chunk, 0)
    plsc.subcore_barrier()

    pltpu.sync_copy(agg_sh.at[pl.ds(base_row, ROWS_PER_TILE)],
                    agg_out.at[c, pl.ds(base_row, ROWS_PER_TILE)])


def _sc_aggregate(src, dst, x):
    mesh = plsc.VectorSubcoreMesh(core_axis_name="c", subcore_axis_name="s")
    return pl.kernel(
        _sc_body,
        out_type=jax.ShapeDtypeStruct((NC, N_PAD, D_IN), jnp.float32),
        mesh=mesh,
        scratch_types=[
            pltpu.VMEM((CHUNK,), jnp.int32),
            pltpu.VMEM((CHUNK,), jnp.int32),
            pltpu.VMEM((CHUNK,), jnp.int32),
            pltpu.VMEM((CHUNK,), jnp.int32),
            pltpu.VMEM((CHUNK, D_IN), jnp.float32),
            pltpu.VMEM((CHUNK, D_IN), jnp.float32),
            pltpu.VMEM_SHARED((N_PAD, D_IN), jnp.float32),
            pltpu.SemaphoreType.DMA,
            pltpu.SemaphoreType.DMA,
            pltpu.SemaphoreType.DMA,
            pltpu.SemaphoreType.DMA,
        ],
        name="sc_edge_aggregate",
    )(src, dst, x)


def _sc_deg_body(dst_hbm, deg_out, didx_v, didx2_v, ones_v, zdeg_v, deg_sh,
                 sem0, sem1):
    c = lax.axis_index("c")
    s = lax.axis_index("s")
    wid = c * NS + s

    _fill_16lane(ones_v, CHUNK, DEG_W, 1.0)
    _fill_16lane(zdeg_v, ZROWS, DEG_W, 0.0)

    base_row = s * ROWS_PER_TILE
    for k in range(ROWS_PER_TILE // ZROWS):
        pltpu.sync_copy(zdeg_v, deg_sh.at[pl.ds(base_row + k * ZROWS, ZROWS)])
    plsc.subcore_barrier()

    tile_base = wid * EPT

    def pair(k, _):
        base_a = tile_base + (2 * k) * CHUNK

        @pl.when(k > 0)
        def _():
            pltpu.make_async_copy(ones_v, deg_sh.at[didx_v], sem0).wait()
        pltpu.sync_copy(dst_hbm.at[pl.ds(base_a, CHUNK)], didx_v)
        pltpu.async_copy(ones_v, deg_sh.at[didx_v], sem0, add=True)

        @pl.when(k > 0)
        def _():
            pltpu.make_async_copy(ones_v, deg_sh.at[didx2_v], sem1).wait()
        pltpu.sync_copy(dst_hbm.at[pl.ds(base_a + CHUNK, CHUNK)], didx2_v)
        pltpu.async_copy(ones_v, deg_sh.at[didx2_v], sem1, add=True)
        return 0

    lax.fori_loop(0, PAIRS, pair, 0)
    pltpu.make_async_copy(ones_v, deg_sh.at[didx_v], sem0).wait()
    pltpu.make_async_copy(ones_v, deg_sh.at[didx2_v], sem1).wait()
    plsc.subcore_barrier()

    pltpu.sync_copy(deg_sh.at[pl.ds(base_row, ROWS_PER_TILE)],
                    deg_out.at[c, pl.ds(base_row, ROWS_PER_TILE)])


def _sc_degree(dst):
    mesh = plsc.VectorSubcoreMesh(core_axis_name="c", subcore_axis_name="s")
    return pl.kernel(
        _sc_deg_body,
        out_type=jax.ShapeDtypeStruct((NC, N_PAD, DEG_W), jnp.float32),
        mesh=mesh,
        scratch_types=[
            pltpu.VMEM((CHUNK,), jnp.int32),
            pltpu.VMEM((CHUNK,), jnp.int32),
            pltpu.VMEM((CHUNK, DEG_W), jnp.float32),
            pltpu.VMEM((ZROWS, DEG_W), jnp.float32),
            pltpu.VMEM_SHARED((N_PAD, DEG_W), jnp.float32),
            pltpu.SemaphoreType.DMA,
            pltpu.SemaphoreType.DMA,
        ],
        name="sc_degree",
    )(dst)


def _tc_body(x_ref, agg_ref, deg_ref, wc_ref, wp_ref, out_ref):
    aggs = agg_ref[0] + agg_ref[1]
    deg = deg_ref[0] + deg_ref[1]
    inv = 1.0 / jnp.maximum(deg, 1.0)
    h = jnp.dot(aggs * inv, wc_ref[...],
                preferred_element_type=jnp.float32)
    out_ref[...] = h + jnp.dot(x_ref[...], wp_ref[...],
                               preferred_element_type=jnp.float32)


def _tc_combine(x, agg_part, deg_part, W_conv, W_proj):
    blk = 1024
    grid = (N_PAD // blk,)
    return pl.pallas_call(
        _tc_body,
        grid=grid,
        in_specs=[
            pl.BlockSpec((blk, D_IN), lambda i: (i, 0)),
            pl.BlockSpec((NC, blk, D_IN), lambda i: (0, i, 0)),
            pl.BlockSpec((NC, blk, DEG_W), lambda i: (0, i, 0)),
            pl.BlockSpec((D_IN, D_OUT), lambda i: (0, 0)),
            pl.BlockSpec((D_IN, D_OUT), lambda i: (0, 0)),
        ],
        out_specs=pl.BlockSpec((blk, D_OUT), lambda i: (i, 0)),
        out_shape=jax.ShapeDtypeStruct((N_NODES, D_OUT), jnp.float32),
    )(x, agg_part, deg_part, W_conv, W_proj)


@jax.jit
def kernel(x, edge_index, W_conv, W_proj):
    src = edge_index[0].reshape(NW, E_PER_W)
    dst = edge_index[1].reshape(NW, E_PER_W)
    npad = EPT - E_PER_W
    src_p = jnp.concatenate(
        [src, jnp.zeros((NW, npad), jnp.int32)], axis=1).reshape(-1)
    dst_p = jnp.concatenate(
        [dst, jnp.full((NW, npad), N_PAD - 1, jnp.int32)], axis=1).reshape(-1)
    agg_part = _sc_aggregate(src_p, dst_p, x)
    deg_part = _sc_degree(dst_p)
    return _tc_combine(x, agg_part, deg_part, W_conv, W_proj)


# speedup vs baseline: 1.1919x; 1.1919x over previous
import jax
import jax.numpy as jnp
from jax import lax
from jax.experimental import pallas as pl
from jax.experimental.pallas import tpu as pltpu
from jax.experimental.pallas import tpu_sc as plsc

N_NODES = 10000
N_EDGES = 320000
D_IN = 128
D_OUT = 256

NC = 2
NS = 16
NW = NC * NS
E_PER_W = N_EDGES // NW
EPT = 10240
CHUNK = 80
N_CHUNKS = EPT // CHUNK
PAIRS = N_CHUNKS // 2
DCHUNK = 128
DPAIRS = EPT // DCHUNK // 2
N_PAD = 10240
ROWS_PER_TILE = N_PAD // NS
ZROWS = 128


DEG_W = 128


def _fill_16lane(ref, nrow, ncol, value):
    nv = ncol // 16
    def row(r, _):
        def col(j, _):
            ref[r, pl.ds(j * 16, 16)] = jnp.full((16,), value, jnp.float32)
            return 0
        return lax.fori_loop(0, nv, col, 0)
    lax.fori_loop(0, nrow, row, 0)


def _sc_body(src_hbm, dst_hbm, x_hbm, agg_out,
             sidx0, sidx1, didx0, didx1, rows0, rows1, agg_sh,
             sem_g0, sem_g1, sem_s0, sem_s1):
    c = lax.axis_index("c")
    s = lax.axis_index("s")
    wid = c * NS + s
    tile_base = wid * EPT

    _fill_16lane(rows0, CHUNK, D_IN, 0.0)
    base_row = s * ROWS_PER_TILE
    for k in range(ROWS_PER_TILE // CHUNK):
        pltpu.sync_copy(rows0, agg_sh.at[pl.ds(base_row + k * CHUNK, CHUNK)])
    plsc.subcore_barrier()

    pltpu.sync_copy(src_hbm.at[pl.ds(tile_base, CHUNK)], sidx0)
    pltpu.async_copy(x_hbm.at[sidx0], rows0, sem_g0)

    def pair(k, _):
        base_a = tile_base + (2 * k) * CHUNK
        base_b = base_a + CHUNK
        pltpu.sync_copy(src_hbm.at[pl.ds(base_b, CHUNK)], sidx1)

        @pl.when(k > 0)
        def _():
            pltpu.make_async_copy(rows1, agg_sh.at[didx1], sem_s1).wait()
        pltpu.async_copy(x_hbm.at[sidx1], rows1, sem_g1)

        pltpu.sync_copy(dst_hbm.at[pl.ds(base_a, CHUNK)], didx0)
        pltpu.make_async_copy(x_hbm.at[sidx0], rows0, sem_g0).wait()
        pltpu.async_copy(rows0, agg_sh.at[didx0], sem_s0, add=True)

        @pl.when(k < PAIRS - 1)
        def _():
            pltpu.sync_copy(src_hbm.at[pl.ds(base_a + 2 * CHUNK, CHUNK)],
                            sidx0)
            pltpu.make_async_copy(rows0, agg_sh.at[didx0], sem_s0).wait()
            pltpu.async_copy(x_hbm.at[sidx0], rows0, sem_g0)

        pltpu.sync_copy(dst_hbm.at[pl.ds(base_b, CHUNK)], didx1)
        pltpu.make_async_copy(x_hbm.at[sidx1], rows1, sem_g1).wait()
        pltpu.async_copy(rows1, agg_sh.at[didx1], sem_s1, add=True)
        return 0

    lax.fori_loop(0, PAIRS, pair, 0)
    pltpu.make_async_copy(rows0, agg_sh.at[didx0], sem_s0).wait()
    pltpu.make_async_copy(rows1, agg_sh.at[didx1], sem_s1).wait()
    plsc.subcore_barrier()

    pltpu.sync_copy(agg_sh.at[pl.ds(base_row, ROWS_PER_TILE)],
                    agg_out.at[c, pl.ds(base_row, ROWS_PER_TILE)])


def _sc_aggregate(src, dst, x):
    mesh = plsc.VectorSubcoreMesh(core_axis_name="c", subcore_axis_name="s")
    return pl.kernel(
        _sc_body,
        out_type=jax.ShapeDtypeStruct((NC, N_PAD, D_IN), jnp.float32),
        mesh=mesh,
        scratch_types=[
            pltpu.VMEM((CHUNK,), jnp.int32),
            pltpu.VMEM((CHUNK,), jnp.int32),
            pltpu.VMEM((CHUNK,), jnp.int32),
            pltpu.VMEM((CHUNK,), jnp.int32),
            pltpu.VMEM((CHUNK, D_IN), jnp.float32),
            pltpu.VMEM((CHUNK, D_IN), jnp.float32),
            pltpu.VMEM_SHARED((N_PAD, D_IN), jnp.float32),
            pltpu.SemaphoreType.DMA,
            pltpu.SemaphoreType.DMA,
            pltpu.SemaphoreType.DMA,
            pltpu.SemaphoreType.DMA,
        ],
        name="sc_edge_aggregate",
    )(src, dst, x)


def _sc_deg_body(dst_hbm, deg_out, didx_v, didx2_v, ones_v, zdeg_v, deg_sh,
                 sem0, sem1):
    c = lax.axis_index("c")
    s = lax.axis_index("s")
    wid = c * NS + s

    _fill_16lane(ones_v, DCHUNK, DEG_W, 1.0)
    _fill_16lane(zdeg_v, ZROWS, DEG_W, 0.0)

    base_row = s * ROWS_PER_TILE
    for k in range(ROWS_PER_TILE // ZROWS):
        pltpu.sync_copy(zdeg_v, deg_sh.at[pl.ds(base_row + k * ZROWS, ZROWS)])
    plsc.subcore_barrier()

    tile_base = wid * EPT

    def pair(k, _):
        base_a = tile_base + (2 * k) * DCHUNK

        @pl.when(k > 0)
        def _():
            pltpu.make_async_copy(ones_v, deg_sh.at[didx_v], sem0).wait()
        pltpu.sync_copy(dst_hbm.at[pl.ds(base_a, DCHUNK)], didx_v)
        pltpu.async_copy(ones_v, deg_sh.at[didx_v], sem0, add=True)

        @pl.when(k > 0)
        def _():
            pltpu.make_async_copy(ones_v, deg_sh.at[didx2_v], sem1).wait()
        pltpu.sync_copy(dst_hbm.at[pl.ds(base_a + DCHUNK, DCHUNK)], didx2_v)
        pltpu.async_copy(ones_v, deg_sh.at[didx2_v], sem1, add=True)
        return 0

    lax.fori_loop(0, DPAIRS, pair, 0)
    pltpu.make_async_copy(ones_v, deg_sh.at[didx_v], sem0).wait()
    pltpu.make_async_copy(ones_v, deg_sh.at[didx2_v], sem1).wait()
    plsc.subcore_barrier()

    pltpu.sync_copy(deg_sh.at[pl.ds(base_row, ROWS_PER_TILE)],
                    deg_out.at[c, pl.ds(base_row, ROWS_PER_TILE)])


def _sc_degree(dst):
    mesh = plsc.VectorSubcoreMesh(core_axis_name="c", subcore_axis_name="s")
    return pl.kernel(
        _sc_deg_body,
        out_type=jax.ShapeDtypeStruct((NC, N_PAD, DEG_W), jnp.float32),
        mesh=mesh,
        scratch_types=[
            pltpu.VMEM((DCHUNK,), jnp.int32),
            pltpu.VMEM((DCHUNK,), jnp.int32),
            pltpu.VMEM((DCHUNK, DEG_W), jnp.float32),
            pltpu.VMEM((ZROWS, DEG_W), jnp.float32),
            pltpu.VMEM_SHARED((N_PAD, DEG_W), jnp.float32),
            pltpu.SemaphoreType.DMA,
            pltpu.SemaphoreType.DMA,
        ],
        name="sc_degree",
    )(dst)


def _tc_body(x_ref, agg_ref, deg_ref, wc_ref, wp_ref, out_ref):
    aggs = agg_ref[0] + agg_ref[1]
    deg = deg_ref[0] + deg_ref[1]
    inv = 1.0 / jnp.maximum(deg, 1.0)
    h = jnp.dot(aggs * inv, wc_ref[...],
                preferred_element_type=jnp.float32)
    out_ref[...] = h + jnp.dot(x_ref[...], wp_ref[...],
                               preferred_element_type=jnp.float32)


def _tc_combine(x, agg_part, deg_part, W_conv, W_proj):
    blk = 1024
    grid = (N_PAD // blk,)
    return pl.pallas_call(
        _tc_body,
        grid=grid,
        in_specs=[
            pl.BlockSpec((blk, D_IN), lambda i: (i, 0)),
            pl.BlockSpec((NC, blk, D_IN), lambda i: (0, i, 0)),
            pl.BlockSpec((NC, blk, DEG_W), lambda i: (0, i, 0)),
            pl.BlockSpec((D_IN, D_OUT), lambda i: (0, 0)),
            pl.BlockSpec((D_IN, D_OUT), lambda i: (0, 0)),
        ],
        out_specs=pl.BlockSpec((blk, D_OUT), lambda i: (i, 0)),
        out_shape=jax.ShapeDtypeStruct((N_NODES, D_OUT), jnp.float32),
    )(x, agg_part, deg_part, W_conv, W_proj)


@jax.jit
def kernel(x, edge_index, W_conv, W_proj):
    src = edge_index[0].reshape(NW, E_PER_W)
    dst = edge_index[1].reshape(NW, E_PER_W)
    npad = EPT - E_PER_W
    src_p = jnp.concatenate(
        [src, jnp.zeros((NW, npad), jnp.int32)], axis=1).reshape(-1)
    dst_p = jnp.concatenate(
        [dst, jnp.full((NW, npad), N_PAD - 1, jnp.int32)], axis=1).reshape(-1)
    agg_part = _sc_aggregate(src_p, dst_p, x)
    deg_part = _sc_degree(dst_p)
    return _tc_combine(x, agg_part, deg_part, W_conv, W_proj)


# speedup vs baseline: 1.7344x; 1.4551x over previous
import jax
import jax.numpy as jnp
from jax import lax
from jax.experimental import pallas as pl
from jax.experimental.pallas import tpu as pltpu
from jax.experimental.pallas import tpu_sc as plsc

N_NODES = 10000
N_EDGES = 320000
D_IN = 128
D_OUT = 256

NC = 2
NS = 16
NW = NC * NS
E_PER_W = N_EDGES // NW
CHUNK = 80
N_CHUNKS = E_PER_W // CHUNK
N_PAD = 10240
ROWS_PER_TILE = N_PAD // NS
ZROWS = 128


DEG_W = 128


def _fill_16lane(ref, nrow, ncol, value):
    nv = ncol // 16
    def row(r, _):
        def col(j, _):
            ref[r, pl.ds(j * 16, 16)] = jnp.full((16,), value, jnp.float32)
            return 0
        return lax.fori_loop(0, nv, col, 0)
    lax.fori_loop(0, nrow, row, 0)


def _sc_body(src_hbm, dst_hbm, x_hbm, agg_out,
             sidx0, didx0, rows0, agg_sh, sem_g0):
    c = lax.axis_index("c")
    s = lax.axis_index("s")
    wid = c * NS + s
    tile_base = wid * E_PER_W

    _fill_16lane(rows0, CHUNK, D_IN, 0.0)
    base_row = s * ROWS_PER_TILE
    for k in range(ROWS_PER_TILE // CHUNK):
        pltpu.sync_copy(rows0, agg_sh.at[pl.ds(base_row + k * CHUNK, CHUNK)])
    plsc.subcore_barrier()

    def chunk(i, _):
        base = tile_base + i * CHUNK
        pltpu.sync_copy(src_hbm.at[pl.ds(base, CHUNK)], sidx0)
        gather = pltpu.async_copy(x_hbm.at[sidx0], rows0, sem_g0)
        pltpu.sync_copy(dst_hbm.at[pl.ds(base, CHUNK)], didx0)
        gather.wait()
        pltpu.sync_copy(rows0, agg_sh.at[didx0], add=True)
        return 0

    lax.fori_loop(0, N_CHUNKS, chunk, 0)
    plsc.subcore_barrier()

    pltpu.sync_copy(agg_sh.at[pl.ds(base_row, ROWS_PER_TILE)],
                    agg_out.at[c, pl.ds(base_row, ROWS_PER_TILE)])


def _sc_aggregate(src, dst, x):
    mesh = plsc.VectorSubcoreMesh(core_axis_name="c", subcore_axis_name="s")
    return pl.kernel(
        _sc_body,
        out_type=jax.ShapeDtypeStruct((NC, N_PAD, D_IN), jnp.float32),
        mesh=mesh,
        scratch_types=[
            pltpu.VMEM((CHUNK,), jnp.int32),
            pltpu.VMEM((CHUNK,), jnp.int32),
            pltpu.VMEM((CHUNK, D_IN), jnp.float32),
            pltpu.VMEM_SHARED((N_PAD, D_IN), jnp.float32),
            pltpu.SemaphoreType.DMA,
        ],
        name="sc_edge_aggregate",
    )(src, dst, x)


def _sc_deg_body(dst_hbm, deg_out, didx_v, didx2_v, ones_v, zdeg_v, deg_sh,
                 sem0, sem1):
    c = lax.axis_index("c")
    s = lax.axis_index("s")
    wid = c * NS + s

    _fill_16lane(ones_v, CHUNK, DEG_W, 1.0)
    _fill_16lane(zdeg_v, ZROWS, DEG_W, 0.0)

    base_row = s * ROWS_PER_TILE
    for k in range(ROWS_PER_TILE // ZROWS):
        pltpu.sync_copy(zdeg_v, deg_sh.at[pl.ds(base_row + k * ZROWS, ZROWS)])
    plsc.subcore_barrier()

    tile_base = wid * E_PER_W
    pltpu.sync_copy(dst_hbm.at[pl.ds(tile_base, CHUNK)], didx_v)
    pltpu.async_copy(ones_v, deg_sh.at[didx_v], sem0, add=True)

    def pair(k, _):
        base_b = tile_base + (2 * k + 1) * CHUNK

        @pl.when(k > 0)
        def _():
            pltpu.make_async_copy(ones_v, deg_sh.at[didx2_v], sem1).wait()
        pltpu.sync_copy(dst_hbm.at[pl.ds(base_b, CHUNK)], didx2_v)
        pltpu.async_copy(ones_v, deg_sh.at[didx2_v], sem1, add=True)

        pltpu.make_async_copy(ones_v, deg_sh.at[didx_v], sem0).wait()
        pltpu.sync_copy(dst_hbm.at[pl.ds(base_b + CHUNK, CHUNK)], didx_v)
        pltpu.async_copy(ones_v, deg_sh.at[didx_v], sem0, add=True)
        return 0

    lax.fori_loop(0, (N_CHUNKS - 1) // 2, pair, 0)
    pltpu.make_async_copy(ones_v, deg_sh.at[didx2_v], sem1).wait()
    pltpu.make_async_copy(ones_v, deg_sh.at[didx_v], sem0).wait()
    plsc.subcore_barrier()

    pltpu.sync_copy(deg_sh.at[pl.ds(base_row, ROWS_PER_TILE)],
                    deg_out.at[c, pl.ds(base_row, ROWS_PER_TILE)])


def _sc_degree(dst):
    mesh = plsc.VectorSubcoreMesh(core_axis_name="c", subcore_axis_name="s")
    return pl.kernel(
        _sc_deg_body,
        out_type=jax.ShapeDtypeStruct((NC, N_PAD, DEG_W), jnp.float32),
        mesh=mesh,
        scratch_types=[
            pltpu.VMEM((CHUNK,), jnp.int32),
            pltpu.VMEM((CHUNK,), jnp.int32),
            pltpu.VMEM((CHUNK, DEG_W), jnp.float32),
            pltpu.VMEM((ZROWS, DEG_W), jnp.float32),
            pltpu.VMEM_SHARED((N_PAD, DEG_W), jnp.float32),
            pltpu.SemaphoreType.DMA,
            pltpu.SemaphoreType.DMA,
        ],
        name="sc_degree",
    )(dst)


def _tc_body(x_ref, agg_ref, deg_ref, wc_ref, wp_ref, out_ref):
    aggs = agg_ref[0] + agg_ref[1]
    deg = deg_ref[0] + deg_ref[1]
    inv = 1.0 / jnp.maximum(deg, 1.0)
    h = jnp.dot(aggs * inv, wc_ref[...],
                preferred_element_type=jnp.float32)
    out_ref[...] = h + jnp.dot(x_ref[...], wp_ref[...],
                               preferred_element_type=jnp.float32)


def _tc_combine(x, agg_part, deg_part, W_conv, W_proj):
    blk = 1024
    grid = (N_PAD // blk,)
    return pl.pallas_call(
        _tc_body,
        grid=grid,
        in_specs=[
            pl.BlockSpec((blk, D_IN), lambda i: (i, 0)),
            pl.BlockSpec((NC, blk, D_IN), lambda i: (0, i, 0)),
            pl.BlockSpec((NC, blk, DEG_W), lambda i: (0, i, 0)),
            pl.BlockSpec((D_IN, D_OUT), lambda i: (0, 0)),
            pl.BlockSpec((D_IN, D_OUT), lambda i: (0, 0)),
        ],
        out_specs=pl.BlockSpec((blk, D_OUT), lambda i: (i, 0)),
        out_shape=jax.ShapeDtypeStruct((N_NODES, D_OUT), jnp.float32),
    )(x, agg_part, deg_part, W_conv, W_proj)


@jax.jit
def kernel(x, edge_index, W_conv, W_proj):
    agg_part = _sc_aggregate(edge_index[0], edge_index[1], x)
    deg_part = _sc_degree(edge_index[1])
    return _tc_combine(x, agg_part, deg_part, W_conv, W_proj)


# speedup vs baseline: 2.3727x; 1.3681x over previous
import jax
import jax.numpy as jnp
from jax import lax
from jax.experimental import pallas as pl
from jax.experimental.pallas import tpu as pltpu
from jax.experimental.pallas import tpu_sc as plsc

N_NODES = 10000
N_EDGES = 320000
D_IN = 128
D_OUT = 256

NC = 2
NS = 16
NW = NC * NS
E_PER_W = N_EDGES // NW
CHUNK = 80
N_CHUNKS = E_PER_W // CHUNK
N_PAD = 10240
ROWS_PER_TILE = N_PAD // NS
ZROWS = 128


DEG_W = 128


def _fill_16lane(ref, nrow, ncol, value):
    nv = ncol // 16
    def row(r, _):
        def col(j, _):
            ref[r, pl.ds(j * 16, 16)] = jnp.full((16,), value, jnp.float32)
            return 0
        return lax.fori_loop(0, nv, col, 0)
    lax.fori_loop(0, nrow, row, 0)


def _sc_body(src_hbm, dst_hbm, x_hbm, agg_out,
             sidx, didx, rows, agg_sh, sem_g, sem_s):
    c = lax.axis_index("c")
    s = lax.axis_index("s")
    wid = c * NS + s
    tile_base = wid * E_PER_W

    _fill_16lane(rows[0], CHUNK, D_IN, 0.0)
    base_row = s * ROWS_PER_TILE
    for k in range(ROWS_PER_TILE // CHUNK):
        pltpu.sync_copy(rows[0], agg_sh.at[pl.ds(base_row + k * CHUNK, CHUNK)])
    plsc.subcore_barrier()

    def start_gather(j, chunk_i):
        pltpu.sync_copy(src_hbm.at[pl.ds(tile_base + chunk_i * CHUNK, CHUNK)],
                        sidx[j])
        pltpu.async_copy(x_hbm.at[sidx[j]], rows[j], sem_g[j])

    def start_scatter(j, chunk_i):
        pltpu.make_async_copy(x_hbm.at[sidx[j]], rows[j], sem_g[j]).wait()
        pltpu.sync_copy(dst_hbm.at[pl.ds(tile_base + chunk_i * CHUNK, CHUNK)],
                        didx[j])
        pltpu.async_copy(rows[j], agg_sh.at[didx[j]], sem_s[j], add=True)

    def wait_scatter(j):
        pltpu.make_async_copy(rows[j], agg_sh.at[didx[j]], sem_s[j]).wait()

    start_gather(0, 0)
    start_gather(1, 1)

    def step(k, _):
        @pl.when(k % 2 == 1)
        def _():
            @pl.when(k >= 3)
            def _():
                wait_scatter(2)
                wait_scatter(3)
            start_gather(2, 2 * k)
            start_gather(3, 2 * k + 1)
            start_scatter(0, 2 * k - 2)
            start_scatter(1, 2 * k - 1)

        @pl.when(k % 2 == 0)
        def _():
            @pl.when(k >= 2)
            def _():
                wait_scatter(0)
                wait_scatter(1)
            start_gather(0, 2 * k)
            start_gather(1, 2 * k + 1)
            start_scatter(2, 2 * k - 2)
            start_scatter(3, 2 * k - 1)
        return 0

    lax.fori_loop(1, 62, step, 0)
    start_scatter(2, 122)
    start_scatter(3, 123)
    wait_scatter(0)
    start_gather(0, 124)
    start_scatter(0, 124)
    wait_scatter(0)
    wait_scatter(1)
    wait_scatter(2)
    wait_scatter(3)
    plsc.subcore_barrier()

    pltpu.sync_copy(agg_sh.at[pl.ds(base_row, ROWS_PER_TILE)],
                    agg_out.at[c, pl.ds(base_row, ROWS_PER_TILE)])


def _sc_aggregate(src, dst, x):
    mesh = plsc.VectorSubcoreMesh(core_axis_name="c", subcore_axis_name="s")
    return pl.kernel(
        _sc_body,
        out_type=jax.ShapeDtypeStruct((NC, N_PAD, D_IN), jnp.float32),
        mesh=mesh,
        scratch_types=[
            [pltpu.VMEM((CHUNK,), jnp.int32)] * 4,
            [pltpu.VMEM((CHUNK,), jnp.int32)] * 4,
            [pltpu.VMEM((CHUNK, D_IN), jnp.float32)] * 4,
            pltpu.VMEM_SHARED((N_PAD, D_IN), jnp.float32),
            [pltpu.SemaphoreType.DMA] * 4,
            [pltpu.SemaphoreType.DMA] * 4,
        ],
        name="sc_edge_aggregate",
    )(src, dst, x)


def _sc_deg_body(dst_hbm, deg_out, didx_v, didx2_v, ones_v, zdeg_v, deg_sh,
                 sem0, sem1):
    c = lax.axis_index("c")
    s = lax.axis_index("s")
    wid = c * NS + s

    _fill_16lane(ones_v, CHUNK, DEG_W, 1.0)
    _fill_16lane(zdeg_v, ZROWS, DEG_W, 0.0)

    base_row = s * ROWS_PER_TILE
    for k in range(ROWS_PER_TILE // ZROWS):
        pltpu.sync_copy(zdeg_v, deg_sh.at[pl.ds(base_row + k * ZROWS, ZROWS)])
    plsc.subcore_barrier()

    tile_base = wid * E_PER_W
    pltpu.sync_copy(dst_hbm.at[pl.ds(tile_base, CHUNK)], didx_v)
    pltpu.async_copy(ones_v, deg_sh.at[didx_v], sem0, add=True)

    def pair(k, _):
        base_b = tile_base + (2 * k + 1) * CHUNK

        @pl.when(k > 0)
        def _():
            pltpu.make_async_copy(ones_v, deg_sh.at[didx2_v], sem1).wait()
        pltpu.sync_copy(dst_hbm.at[pl.ds(base_b, CHUNK)], didx2_v)
        pltpu.async_copy(ones_v, deg_sh.at[didx2_v], sem1, add=True)

        pltpu.make_async_copy(ones_v, deg_sh.at[didx_v], sem0).wait()
        pltpu.sync_copy(dst_hbm.at[pl.ds(base_b + CHUNK, CHUNK)], didx_v)
        pltpu.async_copy(ones_v, deg_sh.at[didx_v], sem0, add=True)
        return 0

    lax.fori_loop(0, (N_CHUNKS - 1) // 2, pair, 0)
    pltpu.make_async_copy(ones_v, deg_sh.at[didx2_v], sem1).wait()
    pltpu.make_async_copy(ones_v, deg_sh.at[didx_v], sem0).wait()
    plsc.subcore_barrier()

    pltpu.sync_copy(deg_sh.at[pl.ds(base_row, ROWS_PER_TILE)],
                    deg_out.at[c, pl.ds(base_row, ROWS_PER_TILE)])


def _sc_degree(dst):
    mesh = plsc.VectorSubcoreMesh(core_axis_name="c", subcore_axis_name="s")
    return pl.kernel(
        _sc_deg_body,
        out_type=jax.ShapeDtypeStruct((NC, N_PAD, DEG_W), jnp.float32),
        mesh=mesh,
        scratch_types=[
            pltpu.VMEM((CHUNK,), jnp.int32),
            pltpu.VMEM((CHUNK,), jnp.int32),
            pltpu.VMEM((CHUNK, DEG_W), jnp.float32),
            pltpu.VMEM((ZROWS, DEG_W), jnp.float32),
            pltpu.VMEM_SHARED((N_PAD, DEG_W), jnp.float32),
            pltpu.SemaphoreType.DMA,
            pltpu.SemaphoreType.DMA,
        ],
        name="sc_degree",
    )(dst)


def _tc_body(x_ref, agg_ref, deg_ref, wc_ref, wp_ref, out_ref):
    aggs = agg_ref[0] + agg_ref[1]
    deg = deg_ref[0] + deg_ref[1]
    inv = 1.0 / jnp.maximum(deg, 1.0)
    h = jnp.dot(aggs * inv, wc_ref[...],
                preferred_element_type=jnp.float32)
    out_ref[...] = h + jnp.dot(x_ref[...], wp_ref[...],
                               preferred_element_type=jnp.float32)


def _tc_combine(x, agg_part, deg_part, W_conv, W_proj):
    blk = 1024
    grid = (N_PAD // blk,)
    return pl.pallas_call(
        _tc_body,
        grid=grid,
        in_specs=[
            pl.BlockSpec((blk, D_IN), lambda i: (i, 0)),
            pl.BlockSpec((NC, blk, D_IN), lambda i: (0, i, 0)),
            pl.BlockSpec((NC, blk, DEG_W), lambda i: (0, i, 0)),
            pl.BlockSpec((D_IN, D_OUT), lambda i: (0, 0)),
            pl.BlockSpec((D_IN, D_OUT), lambda i: (0, 0)),
        ],
        out_specs=pl.BlockSpec((blk, D_OUT), lambda i: (i, 0)),
        out_shape=jax.ShapeDtypeStruct((N_NODES, D_OUT), jnp.float32),
    )(x, agg_part, deg_part, W_conv, W_proj)


@jax.jit
def kernel(x, edge_index, W_conv, W_proj):
    agg_part = _sc_aggregate(edge_index[0], edge_index[1], x)
    deg_part = _sc_degree(edge_index[1])
    return _tc_combine(x, agg_part, deg_part, W_conv, W_proj)
